# baseline (speedup 1.0000x reference)
import jax
import jax.numpy as jnp
from jax import lax
from jax.experimental import pallas as pl
from jax.experimental.pallas import tpu as pltpu
from jax.experimental.pallas import tpu_sc as plsc

VOCAB_SIZE = 1000000
SEQ_LEN = 200
EMBED_DIM = 32
BATCH = 4096

NC = 2
NS = 16
NW = NC * NS

G = 128
NSB = 8
BPH = 4
NCHUNK = SEQ_LEN // NSB

PACKED = VOCAB_SIZE // 4
RSLAB = 64
NSLAB = PACKED // RSLAB
SLAB_ITERS = -(-NSLAB // NW)
TAIL_R = PACKED - NSLAB * RSLAB


def _repack_body(tok_hbm, out_hbm, src_v, out_v):
    wid = lax.axis_index("s") * NC + lax.axis_index("c")

    def slab(j, nrow):
        ntok = nrow * 4
        pltpu.sync_copy(
            tok_hbm.at[pl.ds(j * RSLAB * 4, ntok)], src_v.at[pl.ds(0, ntok)]
        )

        @plsc.parallel_loop(0, ntok, unroll=4)
        def _row(i):
            q = i >> 2
            off = (i & 3) * 32
            for h in range(2):
                out_v[q, pl.ds(off + h * 16, 16)] = src_v[i, pl.ds(h * 16, 16)]

        pltpu.sync_copy(
            out_v.at[pl.ds(0, nrow)], out_hbm.at[pl.ds(j * RSLAB, nrow)]
        )

    @pl.loop(0, SLAB_ITERS)
    def _slab(jj):
        j = jj * NW + wid

        @pl.when(j < NSLAB)
        def _():
            slab(j, RSLAB)

    @pl.when(wid == 0)
    def _tail():
        slab(NSLAB, TAIL_R)


def _gather_body(idxT_hbm, tok4_hbm, posx_hbm, out_hbm,
                 idx_v, idx4_v, bounce_v, out_v, posx_v, sem):
    wid = lax.axis_index("s") * NC + lax.axis_index("c")
    b0 = wid * G
    iota = lax.iota(jnp.int32, 16)

    @pl.loop(0, NCHUNK)
    def _chunk(c):
        s0 = c * NSB
        pltpu.sync_copy(idxT_hbm.at[pl.ds(s0, NSB), pl.ds(b0, G)], idx_v)
        pltpu.sync_copy(posx_hbm.at[pl.ds(s0, NSB)], posx_v)

        for sb in range(NSB):
            for kk in range(G // 16):
                idx4_v[sb, pl.ds(kk * 16, 16)] = (
                    idx_v[sb, pl.ds(kk * 16, 16)] >> 2
                )

        for half in range(NSB // BPH):
            descs = [
                pltpu.async_copy(
                    tok4_hbm.at[idx4_v.at[half * BPH + k]],
                    bounce_v.at[pl.ds(k * G, G)],
                    sem,
                )
                for k in range(BPH)
            ]
            for dd in descs:
                dd.wait()
            for k in range(BPH):
                j = half * BPH + k
                slot = k

                @plsc.parallel_loop(0, EMBED_DIM, unroll=2)
                def _comp(d):
                    diag = iota + d
                    diag = jnp.where(diag >= EMBED_DIM, diag - EMBED_DIM, diag)
                    pv = posx_v[j, pl.ds(d * 16, 16)]
                    sbv = jnp.full((16,), j, jnp.int32)
                    for bb in range(G // 16):
                        raw = idx_v[j, pl.ds(bb * 16, 16)]
                        col = ((raw & 3) << 5) + diag
                        rows = iota + (slot * G + bb * 16)
                        v = plsc.load_gather(bounce_v, [rows, col])
                        plsc.store_scatter(
                            out_v, [sbv, diag, iota + bb * 16], v + pv
                        )

        pltpu.sync_copy(out_v, out_hbm.at[pl.ds(s0, NSB), :, pl.ds(b0, G)])


_MESH = plsc.VectorSubcoreMesh(
    core_axis_name="c", subcore_axis_name="s", num_cores=NC, num_subcores=NS
)
_PARAMS = pltpu.CompilerParams(
    use_tc_tiling_on_sc=True, needs_layout_passes=False
)


@jax.jit
def _sc_embed(idxT, tok, posx):
    tok4 = pl.kernel(
        _repack_body,
        out_type=jax.ShapeDtypeStruct((PACKED, 4 * EMBED_DIM), jnp.float32),
        mesh=_MESH,
        scratch_types=[
            pltpu.VMEM((RSLAB * 4, EMBED_DIM), jnp.float32),
            pltpu.VMEM((RSLAB, 4 * EMBED_DIM), jnp.float32),
        ],
        compiler_params=_PARAMS,
    )(tok)
    return pl.kernel(
        _gather_body,
        out_type=jax.ShapeDtypeStruct((SEQ_LEN, EMBED_DIM, BATCH), jnp.float32),
        mesh=_MESH,
        scratch_types=[
            pltpu.VMEM((NSB, G), jnp.int32),
            pltpu.VMEM((NSB, G), jnp.int32),
            pltpu.VMEM((BPH * G, 128), jnp.float32),
            pltpu.VMEM((NSB, EMBED_DIM, G), jnp.float32),
            pltpu.VMEM((NSB, EMBED_DIM * 16), jnp.float32),
            pltpu.SemaphoreType.DMA,
        ],
        compiler_params=_PARAMS,
    )(idxT, tok4, posx)


def kernel(inputs, token_table, position_table):
    idxT = inputs.astype(jnp.int32).T

    comp = (jnp.arange(EMBED_DIM)[:, None] + jnp.arange(16)[None, :]) % EMBED_DIM
    posx = position_table[:, comp].reshape(SEQ_LEN, EMBED_DIM * 16)
    out = _sc_embed(idxT, token_table, posx)
    return jnp.transpose(out, (2, 0, 1))


# speedup vs baseline: 1.2520x; 1.2520x over previous
import jax
import jax.numpy as jnp
from jax import lax
from jax.experimental import pallas as pl
from jax.experimental.pallas import tpu as pltpu
from jax.experimental.pallas import tpu_sc as plsc

VOCAB_SIZE = 1000000
SEQ_LEN = 200
EMBED_DIM = 32
BATCH = 4096

NC = 2
NS = 16
NW = NC * NS

G = 128
NSB = 8
BPH = 4
NCHUNK = SEQ_LEN // NSB

PACKED = VOCAB_SIZE // 4


def _gather_body(idxT_hbm, tok4_hbm, posx_hbm, out_hbm,
                 idx_v, idx4_v, bounce_v, out_v, posx_v, sems):
    wid = lax.axis_index("s") * NC + lax.axis_index("c")
    b0 = wid * G
    iota = lax.iota(jnp.int32, 16)

    @pl.loop(0, NCHUNK)
    def _chunk(c):
        s0 = c * NSB
        pltpu.sync_copy(idxT_hbm.at[pl.ds(s0, NSB), pl.ds(b0, G)], idx_v)
        pltpu.sync_copy(posx_hbm.at[pl.ds(s0, NSB)], posx_v)

        for sb in range(NSB):
            for kk in range(G // 16):
                idx4_v[sb, pl.ds(kk * 16, 16)] = (
                    idx_v[sb, pl.ds(kk * 16, 16)] >> 2
                )

        def fire(j):
            slot = j % BPH
            return pltpu.async_copy(
                tok4_hbm.at[idx4_v.at[j]],
                bounce_v.at[pl.ds(slot * G, G)],
                sems[slot],
            )

        descs = [fire(j) for j in range(BPH)]
        for j in range(NSB):
            slot = j % BPH
            descs[slot].wait()

            @plsc.parallel_loop(0, EMBED_DIM, unroll=2)
            def _comp(d):
                diag = iota + d
                diag = jnp.where(diag >= EMBED_DIM, diag - EMBED_DIM, diag)
                pv = posx_v[j, pl.ds(d * 16, 16)]
                sbv = jnp.full((16,), j, jnp.int32)
                for bb in range(G // 16):
                    raw = idx_v[j, pl.ds(bb * 16, 16)]
                    col = ((raw & 3) << 5) + diag
                    rows = iota + (slot * G + bb * 16)
                    v = plsc.load_gather(bounce_v, [rows, col])
                    plsc.store_scatter(
                        out_v, [sbv, diag, iota + bb * 16], v + pv
                    )

            if j + BPH < NSB:
                descs[slot] = fire(j + BPH)

        pltpu.sync_copy(out_v, out_hbm.at[pl.ds(s0, NSB), :, pl.ds(b0, G)])


_MESH = plsc.VectorSubcoreMesh(
    core_axis_name="c", subcore_axis_name="s", num_cores=NC, num_subcores=NS
)
_PARAMS = pltpu.CompilerParams(
    use_tc_tiling_on_sc=True, needs_layout_passes=False
)


@jax.jit
def _sc_embed(idxT, tok4, posx):
    return pl.kernel(
        _gather_body,
        out_type=jax.ShapeDtypeStruct((SEQ_LEN, EMBED_DIM, BATCH), jnp.float32),
        mesh=_MESH,
        scratch_types=[
            pltpu.VMEM((NSB, G), jnp.int32),
            pltpu.VMEM((NSB, G), jnp.int32),
            pltpu.VMEM((BPH * G, 128), jnp.float32),
            pltpu.VMEM((NSB, EMBED_DIM, G), jnp.float32),
            pltpu.VMEM((NSB, EMBED_DIM * 16), jnp.float32),
            [pltpu.SemaphoreType.DMA] * BPH,
        ],
        compiler_params=_PARAMS,
    )(idxT, tok4, posx)


def kernel(inputs, token_table, position_table):
    idxT = inputs.astype(jnp.int32).T
    tok4 = token_table.reshape(PACKED, 4 * EMBED_DIM)
    comp = (jnp.arange(EMBED_DIM)[:, None] + jnp.arange(16)[None, :]) % EMBED_DIM
    posx = position_table[:, comp].reshape(SEQ_LEN, EMBED_DIM * 16)
    out = _sc_embed(idxT, tok4, posx)
    return jnp.transpose(out, (2, 0, 1))


# speedup vs baseline: 1.2859x; 1.0271x over previous
import jax
import jax.numpy as jnp
from jax import lax
from jax.experimental import pallas as pl
from jax.experimental.pallas import tpu as pltpu
from jax.experimental.pallas import tpu_sc as plsc

VOCAB_SIZE = 1000000
SEQ_LEN = 200
EMBED_DIM = 32
BATCH = 4096

NC = 2
NS = 16
NW = NC * NS

G = 128
NSB = 8
BPH = 4
NCHUNK = SEQ_LEN // NSB

PACKED = VOCAB_SIZE // 4


def _gather_body(idxT_hbm, tok4_hbm, posx_hbm, out_hbm,
                 idx_v, idx4_v, bounce_v, out_v, posx_v, sems, wsem):
    wid = lax.axis_index("s") * NC + lax.axis_index("c")
    b0 = wid * G
    iota = lax.iota(jnp.int32, 16)

    @pl.loop(0, NCHUNK)
    def _chunk(c):
        s0 = c * NSB
        pltpu.sync_copy(idxT_hbm.at[pl.ds(s0, NSB), pl.ds(b0, G)], idx_v)
        pltpu.sync_copy(posx_hbm.at[pl.ds(s0, NSB)], posx_v)

        for sb in range(NSB):
            for kk in range(G // 16):
                idx4_v[sb, pl.ds(kk * 16, 16)] = (
                    idx_v[sb, pl.ds(kk * 16, 16)] >> 2
                )

        def fire(j):
            slot = j % BPH
            return pltpu.async_copy(
                tok4_hbm.at[idx4_v.at[j]],
                bounce_v.at[pl.ds(slot * G, G)],
                sems[slot],
            )

        descs = [fire(j) for j in range(BPH)]

        @pl.when(c > 0)
        def _():
            pltpu.make_async_copy(
                out_v, out_hbm.at[pl.ds(s0, NSB), :, pl.ds(b0, G)], wsem
            ).wait()

        for j in range(NSB):
            slot = j % BPH
            descs[slot].wait()

            @plsc.parallel_loop(0, EMBED_DIM, unroll=2)
            def _comp(d):
                diag = iota + d
                diag = jnp.where(diag >= EMBED_DIM, diag - EMBED_DIM, diag)
                pv = posx_v[j, pl.ds(d * 16, 16)]
                sbv = jnp.full((16,), j, jnp.int32)
                for bb in range(G // 16):
                    raw = idx_v[j, pl.ds(bb * 16, 16)]
                    col = ((raw & 3) << 5) + diag
                    rows = iota + (slot * G + bb * 16)
                    v = plsc.load_gather(bounce_v, [rows, col])
                    plsc.store_scatter(
                        out_v, [sbv, diag, iota + bb * 16], v + pv
                    )

            if j + BPH < NSB:
                descs[slot] = fire(j + BPH)

        pltpu.async_copy(
            out_v, out_hbm.at[pl.ds(s0, NSB), :, pl.ds(b0, G)], wsem
        )

    pltpu.make_async_copy(
        out_v, out_hbm.at[pl.ds(0, NSB), :, pl.ds(b0, G)], wsem
    ).wait()


_MESH = plsc.VectorSubcoreMesh(
    core_axis_name="c", subcore_axis_name="s", num_cores=NC, num_subcores=NS
)
_PARAMS = pltpu.CompilerParams(
    use_tc_tiling_on_sc=True, needs_layout_passes=False
)


@jax.jit
def _sc_embed(idxT, tok4, posx):
    return pl.kernel(
        _gather_body,
        out_type=jax.ShapeDtypeStruct((SEQ_LEN, EMBED_DIM, BATCH), jnp.float32),
        mesh=_MESH,
        scratch_types=[
            pltpu.VMEM((NSB, G), jnp.int32),
            pltpu.VMEM((NSB, G), jnp.int32),
            pltpu.VMEM((BPH * G, 128), jnp.float32),
            pltpu.VMEM((NSB, EMBED_DIM, G), jnp.float32),
            pltpu.VMEM((NSB, EMBED_DIM * 16), jnp.float32),
            [pltpu.SemaphoreType.DMA] * BPH,
            pltpu.SemaphoreType.DMA,
        ],
        compiler_params=_PARAMS,
    )(idxT, tok4, posx)


def kernel(inputs, token_table, position_table):
    idxT = inputs.astype(jnp.int32).T
    tok4 = token_table.reshape(PACKED, 4 * EMBED_DIM)
    comp = (jnp.arange(EMBED_DIM)[:, None] + jnp.arange(16)[None, :]) % EMBED_DIM
    posx = position_table[:, comp].reshape(SEQ_LEN, EMBED_DIM * 16)
    out = _sc_embed(idxT, tok4, posx)
    return jnp.transpose(out, (2, 0, 1))
